# Initial kernel scaffold; baseline (speedup 1.0000x reference)
import jax
import jax.numpy as jnp
from jax.experimental import pallas as pl


def kernel(X, A, W_g0, b_g0, W_g1, b_g1, W_mlp, b_mlp):
    raise NotImplementedError("write your pallas kernel here")


# speedup vs baseline: 5.9532x; 5.9532x over previous
import jax
import jax.numpy as jnp
from jax.experimental import pallas as pl
from jax.experimental.pallas import tpu as pltpu

ALPHA = 0.05
ROWS = 64
FEAT = 4096
N = 64
TILE = 512
NTILES = FEAT // TILE
BATCH = 8


def _body(xr_ref, a_ref, w0_ref, w1_ref, b0_ref, b1_ref, wm_ref, bm_ref,
          out_ref, h1_ref, h2_ref, s_ref):
    l = pl.program_id(0)
    j = pl.program_id(1)

    @pl.when((l == 0) & (j == 0))
    def _compute_s():
        a = a_ref[...]
        mask = (a != 0.0).astype(jnp.float32)
        ii = jax.lax.broadcasted_iota(jnp.int32, (N, N), 0)
        jj = jax.lax.broadcasted_iota(jnp.int32, (N, N), 1)
        eye = (ii == jj).astype(jnp.float32)
        pad = jnp.float32(N * N) - jnp.sum(mask)
        delta00 = ((ii == 0) & (jj == 0)).astype(jnp.float32)
        cnt = mask + eye + pad * delta00
        deg = jnp.sum(cnt, axis=0, keepdims=True)
        dinv = jax.lax.rsqrt(deg)
        s_ref[...] = cnt.T * dinv * dinv.reshape(N, 1)

    dsj = pl.ds(j * TILE, TILE)

    def _layer(src, w_ref, b_ref, dst_ref):
        t = jnp.dot(src, w_ref[...], preferred_element_type=jnp.float32)
        agg = jnp.dot(s_ref[...], t, preferred_element_type=jnp.float32)
        dst_ref[:, dsj] = ALPHA * xr_ref[:, dsj] + agg + b_ref[:, dsj]

    @pl.when(l == 0)
    def _l0():
        _layer(xr_ref[...], w0_ref, b0_ref, h1_ref)

    @pl.when(l == 1)
    def _l1():
        _layer(h1_ref[...], w1_ref, b1_ref, h2_ref)

    @pl.when((l == 1) & (j == NTILES - 1))
    def _final():
        wm = wm_ref[...]
        bm = bm_ref[...].reshape(64, 1)
        g0 = xr_ref[...].reshape(BATCH, 64, TILE)
        g1 = h1_ref[...].reshape(BATCH, 64, TILE)
        g2 = h2_ref[...].reshape(BATCH, 64, TILE)
        for b in range(BATCH):
            ob = (jnp.dot(wm[:, 0:64], g0[b], preferred_element_type=jnp.float32)
                  + jnp.dot(wm[:, 64:128], g1[b], preferred_element_type=jnp.float32)
                  + jnp.dot(wm[:, 128:192], g2[b], preferred_element_type=jnp.float32)
                  + bm)
            out_ref[pl.ds(b * 8, 8), :] = ob.reshape(8, FEAT)


def kernel(X, A, W_g0, b_g0, W_g1, b_g1, W_mlp, b_mlp):
    batch, c, n, seq = X.shape
    Xr = X.reshape(ROWS, FEAT)
    out_r = pl.pallas_call(
        _body,
        grid=(2, NTILES),
        in_specs=[
            pl.BlockSpec((ROWS, FEAT), lambda l, j: (0, 0)),
            pl.BlockSpec((N, N), lambda l, j: (0, 0)),
            pl.BlockSpec((FEAT, TILE), lambda l, j: (0, jnp.where(l == 0, j, NTILES - 1))),
            pl.BlockSpec((FEAT, TILE), lambda l, j: (0, jnp.where(l == 0, 0, j))),
            pl.BlockSpec((1, FEAT), lambda l, j: (0, 0)),
            pl.BlockSpec((1, FEAT), lambda l, j: (0, 0)),
            pl.BlockSpec((64, 192), lambda l, j: (0, 0)),
            pl.BlockSpec((1, 64), lambda l, j: (0, 0)),
        ],
        out_specs=pl.BlockSpec((ROWS, FEAT), lambda l, j: (0, 0)),
        out_shape=jax.ShapeDtypeStruct((ROWS, FEAT), jnp.float32),
        scratch_shapes=[
            pltpu.VMEM((ROWS, FEAT), jnp.float32),
            pltpu.VMEM((ROWS, FEAT), jnp.float32),
            pltpu.VMEM((N, N), jnp.float32),
        ],
    )(Xr, A, W_g0, W_g1, b_g0.reshape(1, FEAT), b_g1.reshape(1, FEAT),
      W_mlp, b_mlp.reshape(1, 64))
    return out_r.reshape(batch, c, n, seq)
